# baseline (speedup 1.0000x reference)
import functools

import jax
import jax.numpy as jnp
from jax import lax
from jax.experimental import pallas as pl
from jax.experimental.pallas import tpu as pltpu
from jax.experimental.pallas import tpu_sc as plsc

DIM = 32
LANES = 16
TOK = 20
NOUT = 4096 * 26
NC, NS = 2, 16
NW = NC * NS
OUT_PW = NOUT // NW
IDX_PW = OUT_PW * TOK
G = 32
C = G * TOK
NCHUNK = OUT_PW // G
DMA_SLICE = 128
NSLICE = C // DMA_SLICE
EPS = 1e-12
RSQRT_MAGIC = 0x5F3759DF

def _newton_rsqrt(x):
    i = plsc.bitcast(x, jnp.int32)
    i = RSQRT_MAGIC - lax.shift_right_logical(i, 1)
    y = plsc.bitcast(i, jnp.float32)
    xh = x * 0.5
    y = y * (1.5 - xh * y * y)
    y = y * (1.5 - xh * y * y)
    return y


def _make_sc_kernel():
    mesh = plsc.VectorSubcoreMesh(core_axis_name="c", subcore_axis_name="s")

    @functools.partial(
        pl.kernel,
        out_type=jax.ShapeDtypeStruct((NOUT, DIM), jnp.float32),
        mesh=mesh,
        compiler_params=pltpu.CompilerParams(
            needs_layout_passes=False, use_tc_tiling_on_sc=False),
        scratch_types=[
            pltpu.VMEM((IDX_PW,), jnp.int32),
            pltpu.VMEM((C, DIM), jnp.float32),
            pltpu.VMEM((C, DIM), jnp.float32),
            pltpu.VMEM((G, DIM), jnp.float32),
            pltpu.VMEM((G, DIM), jnp.float32),
            pltpu.VMEM((DIM,), jnp.float32),
            pltpu.VMEM((DIM,), jnp.float32),
            pltpu.VMEM((TOK, LANES), jnp.float32),
            pltpu.VMEM((DIM, LANES), jnp.float32),
            pltpu.SemaphoreType.DMA,
            pltpu.SemaphoreType.DMA,
            pltpu.SemaphoreType.DMA,
        ],
    )
    def sc_kernel(ids_hbm, table_hbm, gamma_hbm, beta_hbm, out_hbm,
                  idx_all, rows0, rows1, out0, out1, gam_v, bet_v,
                  invbuf, accbuf, rsem, osem0, osem1):
        wid = lax.axis_index("s") * NC + lax.axis_index("c")
        base_out = wid * OUT_PW
        pltpu.sync_copy(ids_hbm.at[pl.ds(base_out * TOK, IDX_PW)], idx_all)
        pltpu.sync_copy(gamma_hbm, gam_v)
        pltpu.sync_copy(beta_hbm, bet_v)
        lane = lax.iota(jnp.int32, LANES)
        lane_tok = lane * TOK
        zero16 = jnp.zeros((LANES,), jnp.float32)

        rows = (rows0, rows1)
        outs = (out0, out1)
        osems = (osem0, osem1)

        def row_copies(c, b):
            return [
                pltpu.make_async_copy(
                    table_hbm.at[idx_all.at[
                        pl.ds(c * C + j * DMA_SLICE, DMA_SLICE)]],
                    rows[b].at[pl.ds(j * DMA_SLICE, DMA_SLICE)],
                    rsem,
                )
                for j in range(NSLICE)
            ]

        def out_copy(c, b):
            return pltpu.make_async_copy(
                outs[b], out_hbm.at[pl.ds(base_out + c * G, G)], osems[b])

        def compute(b):
            rv = rows[b]
            ov = outs[b]

            def run_pass(xb):
                def stats(l, mi_sum):
                    ridx = lane_tok + (xb * TOK + l)
                    s = zero16
                    q = zero16
                    for d in range(DIM):
                        dcol = jnp.full((LANES,), d, jnp.int32)
                        g = plsc.load_gather(rv, [ridx, dcol])
                        s = s + g
                        q = q + g * g
                    mean = s * (1.0 / DIM)
                    var = q * (1.0 / DIM) - mean * mean
                    inv = _newton_rsqrt(var + EPS)
                    invbuf[l] = inv
                    return mi_sum + mean * inv

                mi_sum = lax.fori_loop(0, TOK, stats, zero16)
                for d in range(DIM):
                    accbuf[d] = zero16

                def wacc(l, carry):
                    ridx = lane_tok + (xb * TOK + l)
                    inv = invbuf[l]
                    for d in range(DIM):
                        dcol = jnp.full((LANES,), d, jnp.int32)
                        g = plsc.load_gather(rv, [ridx, dcol])
                        plsc.addupdate(accbuf.at[d], g * inv)
                    return carry

                lax.fori_loop(0, TOK, wacc, 0)

                grow = lane + xb
                for d in range(DIM):
                    dcol = jnp.full((LANES,), d, jnp.int32)
                    gd = plsc.load_gather(gam_v, [dcol])
                    bd = plsc.load_gather(bet_v, [dcol])
                    val = (accbuf[d] - mi_sum) * gd + bd * float(TOK)
                    plsc.store_scatter(ov, [grow, dcol], val)

            run_pass(0)
            run_pass(LANES)

        def pair(c2, carry):
            for b in (0, 1):
                c = c2 * 2 + b
                for cp in row_copies(c, b):
                    cp.wait()

                @pl.when(c + 1 < NCHUNK)
                def _():
                    for cp in row_copies(c + 1, b ^ 1):
                        cp.start()

                @pl.when(c >= 2)
                def _():
                    out_copy(c - 2, b).wait()

                compute(b)
                out_copy(c, b).start()
            return carry

        for cp in row_copies(0, 0):
            cp.start()
        lax.fori_loop(0, NCHUNK // 2, pair, 0)
        out_copy(NCHUNK - 2, 0).wait()
        out_copy(NCHUNK - 1, 1).wait()

    return sc_kernel


_SC_KERNEL = _make_sc_kernel()


def kernel(input_ids, table, gamma, beta):
    b, f, t = input_ids.shape
    ids_flat = input_ids.reshape(-1)
    out = _SC_KERNEL(ids_flat, table, gamma, beta)
    return out.reshape(b, f, DIM)


# speedup vs baseline: 3.4732x; 3.4732x over previous
import functools

import jax
import jax.numpy as jnp
from jax import lax
from jax.experimental import pallas as pl
from jax.experimental.pallas import tpu as pltpu
from jax.experimental.pallas import tpu_sc as plsc

B, F, TOK = 4096, 26, 20
DIM = 32
LANES = 16
NC, NS = 2, 16
NW = NC * NS
B_PW = B // NW
C = F * TOK
NCHUNK = B_PW
DMA_SLICE = 104
NSLICE = C // DMA_SLICE
EPS = 1e-12
RSQRT_MAGIC = 0x5F3759DF

_GDN = lax.GatherDimensionNumbers(
    offset_dims=(), collapsed_slice_dims=(0,), start_index_map=(0,))


def _lane_perm(v, idx):
    return lax.gather(v, idx.reshape(LANES, 1), _GDN, (1,),
                      mode=lax.GatherScatterMode.PROMISE_IN_BOUNDS)


def _newton_rsqrt(x):
    i = plsc.bitcast(x, jnp.int32)
    i = RSQRT_MAGIC - lax.shift_right_logical(i, 1)
    y = plsc.bitcast(i, jnp.float32)
    y = y * (1.5 - (x * 0.5) * y * y)
    return y


def _make_sc_kernel():
    mesh = plsc.VectorSubcoreMesh(core_axis_name="c", subcore_axis_name="s")

    @functools.partial(
        pl.kernel,
        out_type=jax.ShapeDtypeStruct((B, F, DIM), jnp.float32),
        mesh=mesh,
        compiler_params=pltpu.CompilerParams(
            needs_layout_passes=False, use_tc_tiling_on_sc=False),
        scratch_types=[
            pltpu.VMEM((B_PW * F * TOK,), jnp.int32),
            pltpu.VMEM((C, DIM), jnp.float32),
            pltpu.VMEM((C, DIM), jnp.float32),
            pltpu.VMEM((F, DIM), jnp.float32),
            pltpu.VMEM((F, DIM), jnp.float32),
            pltpu.VMEM((DIM,), jnp.float32),
            pltpu.VMEM((DIM,), jnp.float32),
            pltpu.SemaphoreType.DMA,
            pltpu.SemaphoreType.DMA,
            pltpu.SemaphoreType.DMA,
        ],
    )
    def sc_kernel(ids_hbm, table_hbm, gamma_hbm, beta_hbm, out_hbm,
                  idx_all, rows0, rows1, out0, out1, gam_v, bet_v,
                  rsem, osem0, osem1):
        wid = lax.axis_index("s") * NC + lax.axis_index("c")
        base_b = wid * B_PW
        pltpu.sync_copy(ids_hbm.at[pl.ds(base_b * C, B_PW * C)], idx_all)
        pltpu.sync_copy(gamma_hbm, gam_v)
        pltpu.sync_copy(beta_hbm, bet_v)
        glo = gam_v[pl.ds(0, LANES)]
        ghi = gam_v[pl.ds(LANES, LANES)]
        blo = bet_v[pl.ds(0, LANES)] * float(TOK)
        bhi = bet_v[pl.ds(LANES, LANES)] * float(TOK)
        idx15 = jnp.full((LANES,), LANES - 1, jnp.int32)

        rows = (rows0, rows1)
        outs = (out0, out1)
        osems = (osem0, osem1)

        def row_copies(c, b):
            return [
                pltpu.make_async_copy(
                    table_hbm.at[idx_all.at[
                        pl.ds(c * C + j * DMA_SLICE, DMA_SLICE)]],
                    rows[b].at[pl.ds(j * DMA_SLICE, DMA_SLICE)],
                    rsem,
                )
                for j in range(NSLICE)
            ]

        def out_copy(c, b):
            return pltpu.make_async_copy(
                outs[b], out_hbm.at[base_b + c], osems[b])

        def lane_total(v):
            return _lane_perm(jnp.cumsum(v), idx15)

        def compute(b):
            rv = rows[b]
            ov = outs[b]

            def group(g, gcarry):
                r0 = g * TOK
                acc_lo = jnp.zeros((LANES,), jnp.float32)
                acc_hi = jnp.zeros((LANES,), jnp.float32)
                for l in range(TOK):
                    lo = rv[r0 + l, pl.ds(0, LANES)]
                    hi = rv[r0 + l, pl.ds(LANES, LANES)]
                    tot = lane_total(lo + hi)
                    tot2 = lane_total(lo * lo + hi * hi)
                    mean = tot * (1.0 / DIM)
                    var = tot2 * (1.0 / DIM) - mean * mean
                    inv = _newton_rsqrt(var + EPS)
                    acc_lo = acc_lo + (lo - mean) * inv
                    acc_hi = acc_hi + (hi - mean) * inv
                ov[g, pl.ds(0, LANES)] = acc_lo * glo + blo
                ov[g, pl.ds(LANES, LANES)] = acc_hi * ghi + bhi
                return gcarry

            lax.fori_loop(0, F, group, 0)

        def pair(c2, carry):
            for b in (0, 1):
                c = c2 * 2 + b
                for cp in row_copies(c, b):
                    cp.wait()

                @pl.when(c + 1 < NCHUNK)
                def _():
                    for cp in row_copies(c + 1, b ^ 1):
                        cp.start()

                @pl.when(c >= 2)
                def _():
                    out_copy(c - 2, b).wait()

                compute(b)
                out_copy(c, b).start()
            return carry

        for cp in row_copies(0, 0):
            cp.start()
        lax.fori_loop(0, NCHUNK // 2, pair, 0)
        out_copy(NCHUNK - 2, 0).wait()
        out_copy(NCHUNK - 1, 1).wait()

    return sc_kernel


_SC_KERNEL = _make_sc_kernel()


def kernel(input_ids, table, gamma, beta):
    return _SC_KERNEL(input_ids.reshape(-1), table, gamma, beta)
